# baseline (speedup 1.0000x reference)
import jax
import jax.numpy as jnp
from jax.experimental import pallas as pl
from jax.experimental.pallas import tpu as pltpu


def _round_up(x, m):
    return (x + m - 1) // m * m


_CHUNK = 1024
_TMB = 512


def _make_build_kernel(n_pad, n_chunks):
    def _build_kernel(bounds_ref, lin_ref, out_ref, acc_ref, idx_ref, sem_ref):
        i = pl.program_id(0)
        acc_ref[...] = jnp.zeros_like(acc_ref)
        row0 = i * _TMB
        c0 = bounds_ref[i] // _CHUNK
        c1 = jnp.minimum((bounds_ref[i + 1] + _CHUNK - 1) // _CHUNK, n_chunks)

        lanes = n_pad // 128

        def chunk_body(j, carry):
            cp = pltpu.make_async_copy(lin_ref.at[j], idx_ref, sem_ref)
            cp.start()
            cp.wait()

            def edge_body(k, carry2):
                for u in range(8):
                    linv = idx_ref[0, k * 8 + u]
                    r_loc = (linv // n_pad) - row0
                    sub = r_loc & 7
                    r8 = (r_loc >> 3) << 3
                    c = linv - (linv // n_pad) * n_pad
                    c_hi = c >> 7
                    lane = c & 127
                    valid = jnp.logical_and(r_loc >= 0, r_loc < _TMB)
                    sel_sub = jnp.where(valid, sub, 8)
                    r8c = jnp.clip(r8, 0, _TMB - 8)
                    chi_c = jnp.clip(c_hi, 0, lanes - 1)
                    slab = acc_ref[pl.ds(pl.multiple_of(r8c, 8), 8),
                                   pl.ds(pl.multiple_of(chi_c * 128, 128), 128)]
                    m0 = jax.lax.broadcasted_iota(jnp.int32, (8, 128), 0) == sel_sub
                    m1 = jax.lax.broadcasted_iota(jnp.int32, (8, 128), 1) == lane
                    mask = jnp.logical_and(m0, m1)
                    acc_ref[pl.ds(pl.multiple_of(r8c, 8), 8),
                            pl.ds(pl.multiple_of(chi_c * 128, 128), 128)] = (
                        jnp.where(mask, slab + 1.0, slab))
                return carry2

            jax.lax.fori_loop(0, _CHUNK // 8, edge_body, 0)
            return carry

        jax.lax.fori_loop(c0, c1, chunk_body, 0)
        out_ref[...] = acc_ref[...].astype(jnp.bfloat16)

    return _build_kernel


def _pre_kernel(x_ref, w1l_ref, w1r_ref, b1_ref, y_ref, z_ref):
    x = x_ref[...].astype(jnp.bfloat16)
    y_ref[...] = jnp.dot(x, w1l_ref[...],
                         preferred_element_type=jnp.float32).astype(jnp.bfloat16)
    z_ref[...] = (jnp.dot(x, w1r_ref[...], preferred_element_type=jnp.float32)
                  + b1_ref[...])


def _agg1_kernel(a_ref, y_ref, z_ref, w2l_ref, h_ref, p_ref, inv_ref):
    a = a_ref[...]
    acc = jnp.dot(a, y_ref[...], preferred_element_type=jnp.float32)
    deg = jnp.sum(a.astype(jnp.float32), axis=1, keepdims=True)
    inv = 1.0 / jnp.maximum(deg, 1.0)
    inv_ref[...] = inv
    h = jnp.maximum(acc * inv + z_ref[...], 0.0)
    h_bf = h.astype(jnp.bfloat16)
    h_ref[...] = h_bf
    p_ref[...] = jnp.dot(h_bf, w2l_ref[...],
                         preferred_element_type=jnp.float32).astype(jnp.bfloat16)


def _agg2_kernel(a_ref, p_ref, h_ref, inv_ref, w2r_ref, b2_ref, o_ref):
    acc = jnp.dot(a_ref[...], p_ref[...], preferred_element_type=jnp.float32)
    o_ref[...] = (acc * inv_ref[...]
                  + jnp.dot(h_ref[...], w2r_ref[...],
                            preferred_element_type=jnp.float32)
                  + b2_ref[...])


def kernel(x, edge_index, w1l, w1r, b1, w2l, w2r, b2):
    n, cin = x.shape
    hid = w1l.shape[1]
    cout = w2l.shape[1]
    num_edges = edge_index.shape[1]

    cin_p = _round_up(cin, 128)
    hid_p = _round_up(hid, 128)
    cout_p = _round_up(cout, 128)
    n_pad = _round_up(n, _TMB)
    n_strips = n_pad // _TMB

    src, dst = edge_index[0], edge_index[1]
    lin = jnp.sort(dst * n_pad + src)
    ne_pad = _round_up(num_edges, _CHUNK)
    lin = jnp.pad(lin, (0, ne_pad - num_edges),
                  constant_values=n_pad * n_pad)
    n_chunks = ne_pad // _CHUNK
    bounds = jnp.searchsorted(
        lin, jnp.arange(n_strips + 1, dtype=jnp.int32) * (_TMB * n_pad)
    ).astype(jnp.int32)
    lin3 = lin.reshape(n_chunks, 1, _CHUNK)

    cparams = pltpu.CompilerParams(
        dimension_semantics=("parallel",),
        vmem_limit_bytes=64 * 1024 * 1024,
    )
    const = lambda i: (0, 0)

    a = pl.pallas_call(
        _make_build_kernel(n_pad, n_chunks),
        grid_spec=pltpu.PrefetchScalarGridSpec(
            num_scalar_prefetch=1,
            grid=(n_strips,),
            in_specs=[
                pl.BlockSpec(memory_space=pl.ANY),
            ],
            out_specs=pl.BlockSpec((_TMB, n_pad), lambda i, b: (i, 0)),
            scratch_shapes=[
                pltpu.VMEM((_TMB, n_pad), jnp.float32),
                pltpu.SMEM((1, _CHUNK), jnp.int32),
                pltpu.SemaphoreType.DMA,
            ],
        ),
        out_shape=jax.ShapeDtypeStruct((n_pad, n_pad), jnp.bfloat16),
        compiler_params=cparams,
    )(bounds, lin3)

    x_p = jnp.pad(x, ((0, n_pad - n), (0, cin_p - cin)))
    w1l_p = jnp.pad(w1l, ((0, cin_p - cin), (0, hid_p - hid))).astype(jnp.bfloat16)
    w1r_p = jnp.pad(w1r, ((0, cin_p - cin), (0, hid_p - hid))).astype(jnp.bfloat16)
    b1_p = jnp.pad(b1, ((0, 0), (0, hid_p - hid))).astype(jnp.float32)
    w2l_p = jnp.pad(w2l, ((0, hid_p - hid), (0, cout_p - cout))).astype(jnp.bfloat16)
    w2r_p = jnp.pad(w2r, ((0, hid_p - hid), (0, cout_p - cout))).astype(jnp.bfloat16)
    b2_p = jnp.pad(b2, ((0, 0), (0, cout_p - cout))).astype(jnp.float32)

    y, z = pl.pallas_call(
        _pre_kernel,
        grid=(n_strips,),
        in_specs=[
            pl.BlockSpec((_TMB, cin_p), lambda i: (i, 0)),
            pl.BlockSpec((cin_p, hid_p), const),
            pl.BlockSpec((cin_p, hid_p), const),
            pl.BlockSpec((1, hid_p), const),
        ],
        out_specs=[
            pl.BlockSpec((_TMB, hid_p), lambda i: (i, 0)),
            pl.BlockSpec((_TMB, hid_p), lambda i: (i, 0)),
        ],
        out_shape=[
            jax.ShapeDtypeStruct((n_pad, hid_p), jnp.bfloat16),
            jax.ShapeDtypeStruct((n_pad, hid_p), jnp.float32),
        ],
        compiler_params=cparams,
    )(x_p, w1l_p, w1r_p, b1_p)

    h, p, inv_col = pl.pallas_call(
        _agg1_kernel,
        grid=(n_strips,),
        in_specs=[
            pl.BlockSpec((_TMB, n_pad), lambda i: (i, 0)),
            pl.BlockSpec((n_pad, hid_p), const),
            pl.BlockSpec((_TMB, hid_p), lambda i: (i, 0)),
            pl.BlockSpec((hid_p, cout_p), const),
        ],
        out_specs=[
            pl.BlockSpec((_TMB, hid_p), lambda i: (i, 0)),
            pl.BlockSpec((_TMB, cout_p), lambda i: (i, 0)),
            pl.BlockSpec((_TMB, 1), lambda i: (i, 0)),
        ],
        out_shape=[
            jax.ShapeDtypeStruct((n_pad, hid_p), jnp.bfloat16),
            jax.ShapeDtypeStruct((n_pad, cout_p), jnp.bfloat16),
            jax.ShapeDtypeStruct((n_pad, 1), jnp.float32),
        ],
        compiler_params=cparams,
    )(a, y, z, w2l_p)

    out_p = pl.pallas_call(
        _agg2_kernel,
        grid=(n_strips,),
        in_specs=[
            pl.BlockSpec((_TMB, n_pad), lambda i: (i, 0)),
            pl.BlockSpec((n_pad, cout_p), const),
            pl.BlockSpec((_TMB, hid_p), lambda i: (i, 0)),
            pl.BlockSpec((_TMB, 1), lambda i: (i, 0)),
            pl.BlockSpec((hid_p, cout_p), const),
            pl.BlockSpec((1, cout_p), const),
        ],
        out_specs=pl.BlockSpec((_TMB, cout_p), lambda i: (i, 0)),
        out_shape=jax.ShapeDtypeStruct((n_pad, cout_p), jnp.float32),
        compiler_params=cparams,
    )(a, p, h, inv_col, w2r_p, b2_p)

    return out_p[:n, :cout]


# speedup vs baseline: 1.5316x; 1.5316x over previous
import jax
import jax.numpy as jnp
from jax.experimental import pallas as pl
from jax.experimental.pallas import tpu as pltpu


def _round_up(x, m):
    return (x + m - 1) // m * m


def _pre_kernel(x_ref, w1l_ref, w1r_ref, b1_ref, y_ref, z_ref):
    x = x_ref[...].astype(jnp.bfloat16)
    y_ref[...] = jnp.dot(x, w1l_ref[...],
                         preferred_element_type=jnp.float32).astype(jnp.bfloat16)
    z_ref[...] = (jnp.dot(x, w1r_ref[...], preferred_element_type=jnp.float32)
                  + b1_ref[...])


def _agg1_kernel(a_ref, y_ref, z_ref, w2l_ref, h_ref, p_ref, inv_ref,
                 acc_ref, deg_ref):
    k = pl.program_id(1)

    @pl.when(k == 0)
    def _():
        acc_ref[...] = jnp.zeros_like(acc_ref)
        deg_ref[...] = jnp.zeros_like(deg_ref)

    a = a_ref[...]
    acc_ref[...] += jnp.dot(a, y_ref[...],
                            preferred_element_type=jnp.float32)
    deg_ref[...] += jnp.sum(a.astype(jnp.float32), axis=1, keepdims=True)

    @pl.when(k == pl.num_programs(1) - 1)
    def _():
        inv = 1.0 / jnp.maximum(deg_ref[...], 1.0)
        inv_ref[...] = inv
        h = jnp.maximum(acc_ref[...] * inv + z_ref[...], 0.0)
        h_bf = h.astype(jnp.bfloat16)
        h_ref[...] = h_bf
        p_ref[...] = jnp.dot(h_bf, w2l_ref[...],
                             preferred_element_type=jnp.float32).astype(jnp.bfloat16)


def _agg2_kernel(a_ref, p_ref, h_ref, inv_ref, w2r_ref, b2_ref, o_ref, acc_ref):
    k = pl.program_id(1)

    @pl.when(k == 0)
    def _():
        acc_ref[...] = jnp.zeros_like(acc_ref)

    acc_ref[...] += jnp.dot(a_ref[...], p_ref[...],
                            preferred_element_type=jnp.float32)

    @pl.when(k == pl.num_programs(1) - 1)
    def _():
        o_ref[...] = (acc_ref[...] * inv_ref[...]
                      + jnp.dot(h_ref[...], w2r_ref[...],
                                preferred_element_type=jnp.float32)
                      + b2_ref[...])


def kernel(x, edge_index, w1l, w1r, b1, w2l, w2r, b2):
    n, cin = x.shape
    hid = w1l.shape[1]
    cout = w2l.shape[1]

    cin_p = _round_up(cin, 128)
    hid_p = _round_up(hid, 128)
    cout_p = _round_up(cout, 128)

    tm, tk = 2048, 2048
    n_pad = _round_up(n, tk)
    grid = (n_pad // tm, n_pad // tk)

    src, dst = edge_index[0], edge_index[1]
    lin = dst * n_pad + src
    a = jnp.zeros((n_pad * n_pad,), jnp.bfloat16).at[lin].add(
        1.0).reshape(n_pad, n_pad)

    x_p = jnp.pad(x, ((0, n_pad - n), (0, cin_p - cin)))
    w1l_p = jnp.pad(w1l, ((0, cin_p - cin), (0, hid_p - hid))).astype(jnp.bfloat16)
    w1r_p = jnp.pad(w1r, ((0, cin_p - cin), (0, hid_p - hid))).astype(jnp.bfloat16)
    b1_p = jnp.pad(b1, ((0, 0), (0, hid_p - hid))).astype(jnp.float32)
    w2l_p = jnp.pad(w2l, ((0, hid_p - hid), (0, cout_p - cout))).astype(jnp.bfloat16)
    w2r_p = jnp.pad(w2r, ((0, hid_p - hid), (0, cout_p - cout))).astype(jnp.bfloat16)
    b2_p = jnp.pad(b2, ((0, 0), (0, cout_p - cout))).astype(jnp.float32)

    cparams = pltpu.CompilerParams(
        dimension_semantics=("parallel", "arbitrary"),
        vmem_limit_bytes=64 * 1024 * 1024,
    )
    const = lambda *_: (0, 0)

    y, z = pl.pallas_call(
        _pre_kernel,
        grid=(n_pad // tk,),
        in_specs=[
            pl.BlockSpec((tk, cin_p), lambda i: (i, 0)),
            pl.BlockSpec((cin_p, hid_p), lambda i: (0, 0)),
            pl.BlockSpec((cin_p, hid_p), lambda i: (0, 0)),
            pl.BlockSpec((1, hid_p), lambda i: (0, 0)),
        ],
        out_specs=[
            pl.BlockSpec((tk, hid_p), lambda i: (i, 0)),
            pl.BlockSpec((tk, hid_p), lambda i: (i, 0)),
        ],
        out_shape=[
            jax.ShapeDtypeStruct((n_pad, hid_p), jnp.bfloat16),
            jax.ShapeDtypeStruct((n_pad, hid_p), jnp.float32),
        ],
        compiler_params=pltpu.CompilerParams(
            dimension_semantics=("parallel",),
            vmem_limit_bytes=64 * 1024 * 1024,
        ),
    )(x_p, w1l_p, w1r_p, b1_p)

    h, p, inv_col = pl.pallas_call(
        _agg1_kernel,
        grid=grid,
        in_specs=[
            pl.BlockSpec((tm, tk), lambda i, k: (i, k)),
            pl.BlockSpec((tk, hid_p), lambda i, k: (k, 0)),
            pl.BlockSpec((tm, hid_p), lambda i, k: (i, 0)),
            pl.BlockSpec((hid_p, cout_p), const),
        ],
        out_specs=[
            pl.BlockSpec((tm, hid_p), lambda i, k: (i, 0)),
            pl.BlockSpec((tm, cout_p), lambda i, k: (i, 0)),
            pl.BlockSpec((tm, 1), lambda i, k: (i, 0)),
        ],
        out_shape=[
            jax.ShapeDtypeStruct((n_pad, hid_p), jnp.bfloat16),
            jax.ShapeDtypeStruct((n_pad, cout_p), jnp.bfloat16),
            jax.ShapeDtypeStruct((n_pad, 1), jnp.float32),
        ],
        scratch_shapes=[pltpu.VMEM((tm, hid_p), jnp.float32),
                        pltpu.VMEM((tm, 1), jnp.float32)],
        compiler_params=cparams,
    )(a, y, z, w2l_p)

    out_p = pl.pallas_call(
        _agg2_kernel,
        grid=grid,
        in_specs=[
            pl.BlockSpec((tm, tk), lambda i, k: (i, k)),
            pl.BlockSpec((tk, cout_p), lambda i, k: (k, 0)),
            pl.BlockSpec((tm, hid_p), lambda i, k: (i, 0)),
            pl.BlockSpec((tm, 1), lambda i, k: (i, 0)),
            pl.BlockSpec((hid_p, cout_p), const),
            pl.BlockSpec((1, cout_p), const),
        ],
        out_specs=pl.BlockSpec((tm, cout_p), lambda i, k: (i, 0)),
        out_shape=jax.ShapeDtypeStruct((n_pad, cout_p), jnp.float32),
        scratch_shapes=[pltpu.VMEM((tm, cout_p), jnp.float32)],
        compiler_params=cparams,
    )(a, p, h, inv_col, w2r_p, b2_p)

    return out_p[:n, :cout]
